# baseline (speedup 1.0000x reference)
import math

import jax
import jax.numpy as jnp
from jax.experimental import pallas as pl


def _cmul(ar, ai, br, bi):
    return ar * br - ai * bi, ar * bi + ai * br


def _build_powers(dr, di, e0r, e0i, nrows):
    er, ei = e0r, e0i
    sr, si = dr, di
    rows = 1
    while rows < nrows:
        take = min(rows, nrows - rows)
        nr, ni = _cmul(er[:take], ei[:take], sr, si)
        er = jnp.concatenate([er, nr], axis=0)
        ei = jnp.concatenate([ei, ni], axis=0)
        if 2 * rows < nrows:
            sr, si = _cmul(sr, si, sr, si)
        rows += take
    return er, ei


def _nufft_block_kernel(fc_ref, kx_ref, ky_ref, sdcf_ref, yr_ref, yi_ref):
    G = fc_ref.shape[1] // 2
    tw = -2.0 * math.pi
    ax = tw * kx_ref[0]
    ay = tw * ky_ref[0]
    a2 = jnp.concatenate([ax, ay], axis=0)
    c2 = jnp.cos(a2)
    s2 = jnp.sin(a2)
    dxr, dyr = c2[0:1], c2[1:2]
    dxi, dyi = s2[0:1], s2[1:2]

    one = jnp.ones_like(ax)
    zero = jnp.zeros_like(ax)
    w = sdcf_ref[0]
    cx, sx = _build_powers(dxr, dxi, one, zero, G)
    cy, sy = _build_powers(dyr, dyi, w, zero, G)

    cs = jnp.concatenate([cx, sx], axis=0)
    u = jnp.dot(fc_ref[...], cs, preferred_element_type=jnp.float32)
    ur, ui, vr, vi = u[0:G], u[G:2 * G], u[2 * G:3 * G], u[3 * G:4 * G]
    p = jnp.concatenate([ur * cy, vi * sy, ui * cy, vr * sy], axis=0)
    col = jax.lax.broadcasted_iota(jnp.int32, (8, 4 * G), 1)
    row = jax.lax.broadcasted_iota(jnp.int32, (8, 4 * G), 0)
    z0 = jnp.where(col < G, 1.0, jnp.where(col < 2 * G, -1.0, 0.0))
    z1 = jnp.where(col >= 2 * G, 1.0, 0.0)
    z = jnp.where(row == 0, z0, jnp.where(row == 1, z1, 0.0)).astype(jnp.float32)
    y2 = jnp.dot(z, p, preferred_element_type=jnp.float32)
    yr_ref[0, 0, :] = y2[0]
    yi_ref[0, 0, :] = y2[1]


def _fold_weights(x):
    N = x.shape[0]
    G = N // 2 + 8
    xrt = x[..., 0].T
    xit = x[..., 1].T

    def cfold(m):
        a = m[:, N // 2:]
        b = m[:, N // 2:0:-1]
        zp = jnp.zeros((m.shape[0], G - N // 2 - 1), jnp.float32)
        plus = jnp.concatenate([a + b, m[:, 0:1], zp], axis=1)
        minus = jnp.concatenate([a - b, -m[:, 0:1], zp], axis=1)
        return plus, minus

    def rfold(m):
        a = m[N // 2:, :]
        b = m[N // 2:0:-1, :]
        zp = jnp.zeros((G - N // 2 - 1, G), jnp.float32)
        plus = jnp.concatenate([a + b, m[0:1, :], zp], axis=0)
        minus = jnp.concatenate([a - b, -m[0:1, :], zp], axis=0)
        return plus, minus

    ur, vr = cfold(xrt)
    ui, vi = cfold(xit)
    fpur, fmur = rfold(ur)
    fpui, fmui = rfold(ui)
    fpvr, fmvr = rfold(vr)
    fpvi, fmvi = rfold(vi)
    half_col = jnp.ones((1, G), jnp.float32).at[0, 0].set(0.5)
    half_row = half_col.T
    fpur, fpui, fpvr, fpvi = (f * half_col * half_row
                              for f in (fpur, fpui, fpvr, fpvi))
    fmur, fmui, fmvr, fmvi = (f * half_col for f in (fmur, fmui, fmvr, fmvi))
    fc = jnp.concatenate([
        jnp.concatenate([fpur, -fpvi], axis=1),
        jnp.concatenate([fpui, fpvr], axis=1),
        jnp.concatenate([fmur, -fmvi], axis=1),
        jnp.concatenate([fmui, fmvr], axis=1),
    ], axis=0)
    return fc, G


def kernel(x, trajectory, dcf):
    K = trajectory.shape[1]
    S = 4096 if K % 4096 == 0 else K
    nblk = K // S
    fc, G = _fold_weights(x)
    kx = trajectory[0].reshape(nblk, 1, S)
    ky = trajectory[1].reshape(nblk, 1, S)
    sdcf = jnp.sqrt(dcf).reshape(nblk, 1, S)
    fspec = pl.BlockSpec((4 * G, 2 * G), lambda b: (0, 0))
    rspec = pl.BlockSpec((1, 1, S), lambda b: (b, 0, 0))
    yr, yi = pl.pallas_call(
        _nufft_block_kernel,
        grid=(nblk,),
        in_specs=[fspec] + [rspec] * 3,
        out_specs=[rspec, rspec],
        out_shape=[
            jax.ShapeDtypeStruct((nblk, 1, S), jnp.float32),
            jax.ShapeDtypeStruct((nblk, 1, S), jnp.float32),
        ],
    )(fc, kx, ky, sdcf)
    return jnp.stack([yr.reshape(K), yi.reshape(K)], axis=-1)


# speedup vs baseline: 1.0021x; 1.0021x over previous
import math

import jax
import jax.numpy as jnp
from jax.experimental import pallas as pl
from jax.experimental.pallas import tpu as pltpu


def _cmul(ar, ai, br, bi):
    return ar * br - ai * bi, ar * bi + ai * br


def _build_powers_into(ref, dr, di, e0r, e0i, G):
    ref[pl.ds(0, 1), :] = e0r
    ref[pl.ds(G, 1), :] = e0i
    sr, si = dr, di
    rows = 1
    while rows < G:
        take = min(rows, G - rows)
        ar = ref[pl.ds(0, take), :]
        ai = ref[pl.ds(G, take), :]
        nr, ni = _cmul(ar, ai, sr, si)
        ref[pl.ds(rows, take), :] = nr
        ref[pl.ds(G + rows, take), :] = ni
        if 2 * rows < G:
            sr, si = _cmul(sr, si, sr, si)
        rows += take


def _nufft_block_kernel(fc_ref, kx_ref, ky_ref, sdcf_ref, yr_ref, yi_ref,
                        csx_ref, csy_ref, p_ref):
    G = fc_ref.shape[1] // 2
    tw = -2.0 * math.pi
    ax = tw * kx_ref[0]
    ay = tw * ky_ref[0]
    a2 = jnp.concatenate([ax, ay], axis=0)
    c2 = jnp.cos(a2)
    s2 = jnp.sin(a2)
    dxr, dyr = c2[0:1], c2[1:2]
    dxi, dyi = s2[0:1], s2[1:2]

    one = jnp.ones_like(ax)
    zero = jnp.zeros_like(ax)
    w = sdcf_ref[0]
    _build_powers_into(csx_ref, dxr, dxi, one, zero, G)
    _build_powers_into(csy_ref, dyr, dyi, w, zero, G)

    u = jnp.dot(fc_ref[...], csx_ref[...], preferred_element_type=jnp.float32)
    cy = csy_ref[pl.ds(0, G), :]
    sy = csy_ref[pl.ds(G, G), :]
    p_ref[pl.ds(0, G), :] = u[0:G] * cy
    p_ref[pl.ds(G, G), :] = u[3 * G:4 * G] * sy
    p_ref[pl.ds(2 * G, G), :] = u[G:2 * G] * cy
    p_ref[pl.ds(3 * G, G), :] = u[2 * G:3 * G] * sy
    col = jax.lax.broadcasted_iota(jnp.int32, (8, 4 * G), 1)
    row = jax.lax.broadcasted_iota(jnp.int32, (8, 4 * G), 0)
    z0 = jnp.where(col < G, 1.0, jnp.where(col < 2 * G, -1.0, 0.0))
    z1 = jnp.where(col >= 2 * G, 1.0, 0.0)
    z = jnp.where(row == 0, z0, jnp.where(row == 1, z1, 0.0)).astype(jnp.float32)
    y2 = jnp.dot(z, p_ref[...], preferred_element_type=jnp.float32)
    yr_ref[0, 0, :] = y2[0]
    yi_ref[0, 0, :] = y2[1]


def _fold_weights(x):
    N = x.shape[0]
    G = N // 2 + 8
    xrt = x[..., 0].T
    xit = x[..., 1].T

    def cfold(m):
        a = m[:, N // 2:]
        b = m[:, N // 2:0:-1]
        zp = jnp.zeros((m.shape[0], G - N // 2 - 1), jnp.float32)
        plus = jnp.concatenate([a + b, m[:, 0:1], zp], axis=1)
        minus = jnp.concatenate([a - b, -m[:, 0:1], zp], axis=1)
        return plus, minus

    def rfold(m):
        a = m[N // 2:, :]
        b = m[N // 2:0:-1, :]
        zp = jnp.zeros((G - N // 2 - 1, G), jnp.float32)
        plus = jnp.concatenate([a + b, m[0:1, :], zp], axis=0)
        minus = jnp.concatenate([a - b, -m[0:1, :], zp], axis=0)
        return plus, minus

    ur, vr = cfold(xrt)
    ui, vi = cfold(xit)
    fpur, fmur = rfold(ur)
    fpui, fmui = rfold(ui)
    fpvr, fmvr = rfold(vr)
    fpvi, fmvi = rfold(vi)
    half_col = jnp.ones((1, G), jnp.float32).at[0, 0].set(0.5)
    half_row = half_col.T
    fpur, fpui, fpvr, fpvi = (f * half_col * half_row
                              for f in (fpur, fpui, fpvr, fpvi))
    fmur, fmui, fmvr, fmvi = (f * half_col for f in (fmur, fmui, fmvr, fmvi))
    fc = jnp.concatenate([
        jnp.concatenate([fpur, -fpvi], axis=1),
        jnp.concatenate([fpui, fpvr], axis=1),
        jnp.concatenate([fmur, -fmvi], axis=1),
        jnp.concatenate([fmui, fmvr], axis=1),
    ], axis=0)
    return fc, G


def kernel(x, trajectory, dcf):
    K = trajectory.shape[1]
    S = 4096 if K % 4096 == 0 else K
    nblk = K // S
    fc, G = _fold_weights(x)
    kx = trajectory[0].reshape(nblk, 1, S)
    ky = trajectory[1].reshape(nblk, 1, S)
    sdcf = jnp.sqrt(dcf).reshape(nblk, 1, S)
    fspec = pl.BlockSpec((4 * G, 2 * G), lambda b: (0, 0))
    rspec = pl.BlockSpec((1, 1, S), lambda b: (b, 0, 0))
    yr, yi = pl.pallas_call(
        _nufft_block_kernel,
        grid=(nblk,),
        in_specs=[fspec] + [rspec] * 3,
        out_specs=[rspec, rspec],
        out_shape=[
            jax.ShapeDtypeStruct((nblk, 1, S), jnp.float32),
            jax.ShapeDtypeStruct((nblk, 1, S), jnp.float32),
        ],
        scratch_shapes=[
            pltpu.VMEM((2 * G, S), jnp.float32),
            pltpu.VMEM((2 * G, S), jnp.float32),
            pltpu.VMEM((4 * G, S), jnp.float32),
        ],
    )(fc, kx, ky, sdcf)
    return jnp.stack([yr.reshape(K), yi.reshape(K)], axis=-1)
